# baseline (speedup 1.0000x reference)
import functools

import jax
import jax.numpy as jnp
from jax import lax
from jax.experimental import pallas as pl
from jax.experimental.pallas import tpu as pltpu
from jax.experimental.pallas import tpu_sc as plsc

BATCH = 4096
HIST = 50
EMBED_DIM = 128
NUM_CORES = 2
NUM_SUBCORES = 16
NW = NUM_CORES * NUM_SUBCORES
PER_W = BATCH // NW
CHUNK = 64
SPLIT = PER_W // CHUNK
NCH = HIST * SPLIT
GRP = 5
NMACRO = NCH // GRP


def _gather_rows(table, idx_t):
    mesh = plsc.VectorSubcoreMesh(core_axis_name="c", subcore_axis_name="s")

    scratch = (
        [pltpu.VMEM((HIST, PER_W), jnp.int32)]
        + [pltpu.VMEM((CHUNK, EMBED_DIM), jnp.float32) for _ in range(2 * GRP)]
        + [pltpu.SemaphoreType.DMA for _ in range(4 * GRP)]
    )

    @functools.partial(
        pl.kernel,
        mesh=mesh,
        out_type=jax.ShapeDtypeStruct((HIST, BATCH, EMBED_DIM), jnp.float32),
        scratch_types=scratch,
    )
    def k(table_hbm, idx_hbm, out_hbm, idx_v, *bufs_and_sems):
        rows = bufs_and_sems[: 2 * GRP]
        gsem = bufs_and_sems[2 * GRP: 4 * GRP]
        osem = bufs_and_sems[4 * GRP:]

        wid = lax.axis_index("s") * NUM_CORES + lax.axis_index("c")
        c0 = wid * PER_W
        pltpu.sync_copy(idx_hbm.at[:, pl.ds(c0, PER_W)], idx_v)

        def idx_ref(chunk):
            t = chunk // SPLIT
            h = chunk % SPLIT
            return idx_v.at[t, pl.ds(h * CHUNK, CHUNK)]

        def out_ref(chunk):
            t = chunk // SPLIT
            h = chunk % SPLIT
            return out_hbm.at[t, pl.ds(c0 + h * CHUNK, CHUNK)]

        def fire_gather(slot, chunk):
            pltpu.async_copy(table_hbm.at[idx_ref(chunk)], rows[slot],
                             gsem[slot])

        def wait_gather(slot, chunk):
            pltpu.make_async_copy(table_hbm.at[idx_ref(chunk)], rows[slot],
                                  gsem[slot]).wait()

        def fire_out(slot, chunk):
            pltpu.async_copy(rows[slot], out_ref(chunk), osem[slot])

        def wait_out(slot, chunk):
            pltpu.make_async_copy(rows[slot], out_ref(chunk),
                                  osem[slot]).wait()

        def slots(group):
            return range(group * GRP, group * GRP + GRP)

        def phase_a(group, base):
            for j, s in enumerate(slots(group)):
                wait_gather(s, base + j)
                fire_out(s, base + j)

        def phase_b(group, prev_base, next_base):
            for j, s in enumerate(slots(group)):
                wait_out(s, prev_base + j)
                fire_gather(s, next_base + j)

        for j, s in enumerate(slots(0)):
            fire_gather(s, j)
        for j, s in enumerate(slots(1)):
            fire_gather(s, GRP + j)

        phase_a(0, 0)

        def body(kk, carry):
            m1 = 2 * kk + 1
            phase_a(1, m1 * GRP)
            phase_b(0, (m1 - 1) * GRP, (m1 + 1) * GRP)
            m2 = m1 + 1
            phase_a(0, m2 * GRP)
            phase_b(1, (m2 - 1) * GRP, (m2 + 1) * GRP)
            return carry

        lax.fori_loop(0, (NMACRO - 2) // 2, body, 0)

        last = (NMACRO - 1) * GRP
        phase_a(1, last)
        for j, s in enumerate(slots(0)):
            wait_out(s, last - GRP + j)
        for j, s in enumerate(slots(1)):
            wait_out(s, last + j)

    return k(table, idx_t)


@jax.jit
def kernel(indices, table):
    idx_t = indices.astype(jnp.int32).T
    out_t = _gather_rows(table, idx_t)
    return out_t.transpose(1, 0, 2)


# speedup vs baseline: 1.0119x; 1.0119x over previous
import functools

import jax
import jax.numpy as jnp
from jax import lax
from jax.experimental import pallas as pl
from jax.experimental.pallas import tpu as pltpu
from jax.experimental.pallas import tpu_sc as plsc

BATCH = 4096
HIST = 50
EMBED_DIM = 128
NUM_CORES = 2
NUM_SUBCORES = 16
NW = NUM_CORES * NUM_SUBCORES
PER_W = BATCH // NW
CHUNK = 64
SPLIT = PER_W // CHUNK
NCH = HIST * SPLIT
GRP = 5
NMACRO = NCH // GRP


def _gather_rows(table, idx_t):
    mesh = plsc.VectorSubcoreMesh(core_axis_name="c", subcore_axis_name="s")

    scratch = (
        [pltpu.VMEM((HIST, PER_W), jnp.int32)]
        + [pltpu.VMEM((CHUNK, EMBED_DIM), jnp.float32) for _ in range(2 * GRP)]
        + [pltpu.SemaphoreType.DMA for _ in range(4 * GRP)]
    )

    @functools.partial(
        pl.kernel,
        mesh=mesh,
        out_type=jax.ShapeDtypeStruct((HIST, BATCH, EMBED_DIM), jnp.float32),
        scratch_types=scratch,
    )
    def k(table_hbm, idx_hbm, out_hbm, idx_v, *bufs_and_sems):
        rows = bufs_and_sems[: 2 * GRP]
        gsem = bufs_and_sems[2 * GRP: 4 * GRP]
        osem = bufs_and_sems[4 * GRP:]

        wid = lax.axis_index("s") * NUM_CORES + lax.axis_index("c")
        c0 = wid * PER_W
        pltpu.sync_copy(idx_hbm.at[:, pl.ds(c0, PER_W)], idx_v)

        def idx_ref(chunk):
            t = chunk // SPLIT
            h = chunk % SPLIT
            return idx_v.at[t, pl.ds(h * CHUNK, CHUNK)]

        def out_ref(chunk):
            t = chunk // SPLIT
            h = chunk % SPLIT
            return out_hbm.at[t, pl.ds(c0 + h * CHUNK, CHUNK)]

        def fire_gather(slot, chunk):
            pltpu.async_copy(table_hbm.at[idx_ref(chunk)], rows[slot],
                             gsem[slot])

        def wait_gather(slot, chunk):
            pltpu.make_async_copy(table_hbm.at[idx_ref(chunk)], rows[slot],
                                  gsem[slot]).wait()

        def fire_out(slot, chunk):
            pltpu.async_copy(rows[slot], out_ref(chunk), osem[slot])

        def wait_out(slot, chunk):
            pltpu.make_async_copy(rows[slot], out_ref(chunk),
                                  osem[slot]).wait()

        def slots(group):
            return range(group * GRP, group * GRP + GRP)

        def phase_a(group, base):
            for j, s in enumerate(slots(group)):
                wait_gather(s, base + j)
                fire_out(s, base + j)

        def phase_b(group, prev_base, next_base):
            for j, s in enumerate(slots(group)):
                wait_out(s, prev_base + j)
                fire_gather(s, next_base + j)

        for j, s in enumerate(slots(0)):
            fire_gather(s, j)
        for j, s in enumerate(slots(1)):
            fire_gather(s, GRP + j)

        phase_a(0, 0)

        def phase_ab(act, idle, base):
            for j in range(GRP):
                wait_out(idle * GRP + j, base - GRP + j)
                fire_gather(idle * GRP + j, base + GRP + j)
                wait_gather(act * GRP + j, base + j)
                fire_out(act * GRP + j, base + j)

        def body(kk, carry):
            m1 = 2 * kk + 1
            phase_ab(1, 0, m1 * GRP)
            m2 = m1 + 1
            phase_ab(0, 1, m2 * GRP)
            return carry

        lax.fori_loop(0, (NMACRO - 2) // 2, body, 0)

        last = (NMACRO - 1) * GRP
        phase_a(1, last)
        for j, s in enumerate(slots(0)):
            wait_out(s, last - GRP + j)
        for j, s in enumerate(slots(1)):
            wait_out(s, last + j)

    return k(table, idx_t)


@jax.jit
def kernel(indices, table):
    idx_t = indices.astype(jnp.int32).T
    out_t = _gather_rows(table, idx_t)
    return out_t.transpose(1, 0, 2)
